# baseline (speedup 1.0000x reference)
import jax
import jax.numpy as jnp
from jax.experimental import pallas as pl
from jax.experimental.pallas import tpu as pltpu

_TB = 2
_NBUF = 4
_CH = 16


def _fwc_body(s_ref, wt_ref, uft_ref, out_hbm, uw_ref, stage_ref, sems):
    e_total = wt_ref.shape[0]
    rows_per_slot = stage_ref.shape[1]
    rows_per_chunk = rows_per_slot // _CH
    nb = pl.num_programs(0)
    j = pl.program_id(0)

    @pl.when(j == 0)
    def _():
        for e in range(e_total):
            uw_ref[e] = jnp.dot(
                wt_ref[e], uft_ref[...], preferred_element_type=jnp.float32
            )

    slot = jax.lax.rem(j, _NBUF)

    def wait_slot(jq, sq):
        base = jq * rows_per_slot
        for c in range(_CH):
            pltpu.make_async_copy(
                stage_ref.at[sq, pl.ds(c * rows_per_chunk, rows_per_chunk)],
                out_hbm.at[pl.ds(base + c * rows_per_chunk, rows_per_chunk)],
                sems.at[sq],
            ).wait()

    @pl.when(j >= _NBUF)
    def _():
        wait_slot(j - _NBUF, slot)

    base = j * rows_per_slot
    for c in range(_CH):
        pltpu.make_async_copy(
            stage_ref.at[slot, pl.ds(c * rows_per_chunk, rows_per_chunk)],
            out_hbm.at[pl.ds(base + c * rows_per_chunk, rows_per_chunk)],
            sems.at[slot],
        ).start()

    @pl.when(j == nb - 1)
    def _():
        for d in range(_NBUF):
            jd = j - d
            sd = jax.lax.rem(jd, _NBUF)

            @pl.when(jd >= 0)
            def _():
                wait_slot(jd, sd)


def kernel(U, W, node_attributes):
    M, N1, N2, N3, K = U.shape
    E, _, C = W.shape
    B = node_attributes.shape[0]
    X = M * N1 * N2 * N3

    uft = U.reshape(X, K).T.astype(jnp.float32)
    wt = W.transpose(0, 2, 1).astype(jnp.float32)
    species = jnp.argmax(node_attributes, axis=1).astype(jnp.int32)

    nb = B // _TB
    out = pl.pallas_call(
        _fwc_body,
        out_shape=jax.ShapeDtypeStruct((B * C, X), jnp.float32),
        grid_spec=pltpu.PrefetchScalarGridSpec(
            num_scalar_prefetch=1,
            grid=(nb,),
            in_specs=[
                pl.BlockSpec((E, C, K), lambda j, s: (0, 0, 0)),
                pl.BlockSpec((K, X), lambda j, s: (0, 0)),
            ],
            out_specs=pl.BlockSpec(memory_space=pl.ANY),
            scratch_shapes=[
                pltpu.VMEM((E, C, X), jnp.float32),
                pltpu.VMEM((_NBUF, _TB * C, X), jnp.float32),
                pltpu.SemaphoreType.DMA((_NBUF,)),
            ],
        ),
        compiler_params=pltpu.CompilerParams(
            dimension_semantics=("arbitrary",),
            vmem_limit_bytes=52 * 1024 * 1024,
        ),
        name="fwc_probe_chunks",
    )(species, wt, uft)
    return out.reshape(B, C, M, N1, N2, N3)


# speedup vs baseline: 2.2140x; 2.2140x over previous
import jax
import jax.numpy as jnp
from jax.experimental import pallas as pl
from jax.experimental.pallas import tpu as pltpu

_TB = 4


def _fwc_body(s_ref, wt_ref, uft_ref, out_ref, uw_ref):
    e_total = wt_ref.shape[0]
    j = pl.program_id(0)

    @pl.when(j == 0)
    def _():
        for e in range(e_total):
            uw_ref[e] = jnp.dot(
                wt_ref[e], uft_ref[...], preferred_element_type=jnp.float32
            )

    for t in range(_TB):
        out_ref[t] = uw_ref[s_ref[j * _TB + t]]


def kernel(U, W, node_attributes):
    M, N1, N2, N3, K = U.shape
    E, _, C = W.shape
    B = node_attributes.shape[0]
    X = M * N1 * N2 * N3

    uft = U.reshape(X, K).T.astype(jnp.float32)
    wt = W.transpose(0, 2, 1).astype(jnp.float32)
    species = jnp.argmax(node_attributes, axis=1).astype(jnp.int32)

    nb = B // _TB
    out = pl.pallas_call(
        _fwc_body,
        out_shape=jax.ShapeDtypeStruct((B, C, X), jnp.float32),
        grid_spec=pltpu.PrefetchScalarGridSpec(
            num_scalar_prefetch=1,
            grid=(nb,),
            in_specs=[
                pl.BlockSpec((E, C, K), lambda j, s: (0, 0, 0)),
                pl.BlockSpec((K, X), lambda j, s: (0, 0)),
            ],
            out_specs=pl.BlockSpec((_TB, C, X), lambda j, s: (j, 0, 0)),
            scratch_shapes=[pltpu.VMEM((E, C, X), jnp.float32)],
        ),
        compiler_params=pltpu.CompilerParams(
            dimension_semantics=("arbitrary",),
            vmem_limit_bytes=48 * 1024 * 1024,
        ),
        name="fwc_gather",
    )(species, wt, uft)
    return out.reshape(B, C, M, N1, N2, N3)


# speedup vs baseline: 8.3318x; 3.7631x over previous
import jax
import jax.numpy as jnp
from jax.experimental import pallas as pl
from jax.experimental.pallas import tpu as pltpu

_TB = 4


def _fwc_body(s_ref, uf_ref, w_ref, out_ref, uw_ref):
    e_total = w_ref.shape[0]
    j = pl.program_id(0)

    @pl.when(j == 0)
    def _():
        for e in range(e_total):
            uw_ref[e] = jnp.dot(
                uf_ref[...], w_ref[e], preferred_element_type=jnp.float32
            )

    for t in range(_TB):
        out_ref[t] = uw_ref[s_ref[j * _TB + t]]


def kernel(U, W, node_attributes):
    M, N1, N2, N3, K = U.shape
    E, _, C = W.shape
    B = node_attributes.shape[0]
    X = M * N1 * N2 * N3

    uf = U.reshape(X, K).astype(jnp.float32)
    w = W.astype(jnp.float32)
    species = jnp.argmax(node_attributes, axis=1).astype(jnp.int32)

    nb = B // _TB
    out = pl.pallas_call(
        _fwc_body,
        out_shape=jax.ShapeDtypeStruct((B, X, C), jnp.float32),
        grid_spec=pltpu.PrefetchScalarGridSpec(
            num_scalar_prefetch=1,
            grid=(nb,),
            in_specs=[
                pl.BlockSpec((X, K), lambda j, s: (0, 0)),
                pl.BlockSpec((E, K, C), lambda j, s: (0, 0, 0)),
            ],
            out_specs=pl.BlockSpec((_TB, X, C), lambda j, s: (j, 0, 0)),
            scratch_shapes=[pltpu.VMEM((E, X, C), jnp.float32)],
        ),
        compiler_params=pltpu.CompilerParams(
            dimension_semantics=("arbitrary",),
            vmem_limit_bytes=48 * 1024 * 1024,
        ),
        name="fwc_gather",
    )(species, uf, w)
    return out.transpose(0, 2, 1).reshape(B, C, M, N1, N2, N3)
